# Initial kernel scaffold; baseline (speedup 1.0000x reference)
import jax
import jax.numpy as jnp
from jax.experimental import pallas as pl


def kernel(x_spot, x_user, ei_user_spot, ei_spot_user, W_src_us, W_tgt_us, W_src_su, W_tgt_su, Wih_us, Whh_us, bih_us, bhh_us, Wih_su, Whh_su, bih_su, bhh_su):
    raise NotImplementedError("write your pallas kernel here")


# speedup vs baseline: 7.0498x; 7.0498x over previous
import jax
import jax.numpy as jnp
from jax import lax
from jax.experimental import pallas as pl
from jax.experimental.pallas import tpu as pltpu
from jax.experimental.pallas import tpu_sc as plsc

N = 10000
DIN = 128
H = 128
E = 640000
WIDE = 144
CNT = 128
N_PAD = 10240
NS = 16
CHUNK = 64
KB = 5
NBLK = 125
ROWS_PER_TILE = N_PAD // NS
BLK = 1000


def _proj_body(x_ref, w_ref, o_ref):
    x = x_ref[0]
    w = w_ref[0]
    xw = lax.dot_general(x, w, (((1,), (1,)), ((), ())),
                         preferred_element_type=jnp.float32)
    extra = jnp.where(
        lax.broadcasted_iota(jnp.int32, (BLK, WIDE - H), 1) == 0, 1.0, 0.0)
    o_ref[0] = jnp.concatenate([xw, extra], axis=1)


def _sc_body(table_hbm, idx_hbm, zros_hbm, out_hbm,
             sidx_v, didx_v, rows_v, acc_sh, sem):
    cid = lax.axis_index("c")
    tid = lax.axis_index("s")
    base = tid * ROWS_PER_TILE
    pltpu.sync_copy(zros_hbm, rows_v)
    for k in range(ROWS_PER_TILE // CHUNK):
        pltpu.sync_copy(rows_v, acc_sh.at[pl.ds(base + k * CHUNK, CHUNK)])
    plsc.subcore_barrier()
    ws = cid * NS + tid
    wd = 2 * NS + ws

    def block(b, carry):
        pltpu.sync_copy(idx_hbm.at[ws, b], sidx_v)
        pltpu.sync_copy(idx_hbm.at[wd, b], didx_v)
        for k in range(KB):
            pltpu.async_copy(table_hbm.at[sidx_v.at[k]], rows_v, sem).wait()
            pltpu.sync_copy(rows_v, acc_sh.at[didx_v.at[k]], add=True)
        return carry

    lax.fori_loop(0, NBLK, block, 0, unroll=False)
    plsc.subcore_barrier()
    for k in range(ROWS_PER_TILE // CHUNK):
        sl = pl.ds(base + k * CHUNK, CHUNK)
        pltpu.sync_copy(acc_sh.at[sl], rows_v)
        pltpu.sync_copy(rows_v, out_hbm.at[cid].at[sl])


def _gru_body(x_ref, wt_ref, acc_ref, wih_ref, whh_ref, bih_ref, bhh_ref,
              o_ref):
    x = x_ref[0]
    tgt = lax.dot_general(x, wt_ref[0], (((1,), (1,)), ((), ())),
                          preferred_element_type=jnp.float32)
    acc = acc_ref[0]
    agg = acc[:, :H] / jnp.maximum(acc[:, CNT:CNT + 1], 1.0)
    gi = lax.dot_general(tgt, wih_ref[0], (((1,), (1,)), ((), ())),
                         preferred_element_type=jnp.float32) + bih_ref[0, 0]
    gh = lax.dot_general(agg, whh_ref[0], (((1,), (1,)), ((), ())),
                         preferred_element_type=jnp.float32) + bhh_ref[0, 0]
    r = jax.nn.sigmoid(gi[:, :H] + gh[:, :H])
    z = jax.nn.sigmoid(gi[:, H:2 * H] + gh[:, H:2 * H])
    n = jnp.tanh(gi[:, 2 * H:] + r * gh[:, 2 * H:])
    o_ref[0] = jax.nn.relu((1.0 - z) * n + z * agg)


def kernel(x_spot, x_user, ei_user_spot, ei_spot_user,
           W_src_us, W_tgt_us, W_src_su, W_tgt_su,
           Wih_us, Whh_us, bih_us, bhh_us,
           Wih_su, Whh_su, bih_su, bhh_su):
    f32 = jnp.float32
    nb = N // BLK

    x_src = jnp.stack([x_user, x_spot])
    w_src = jnp.stack([W_src_us, W_src_su])
    table = pl.pallas_call(
        _proj_body,
        grid=(2, nb),
        in_specs=[
            pl.BlockSpec((1, BLK, DIN), lambda b, i: (b, i, 0)),
            pl.BlockSpec((1, H, DIN), lambda b, i: (b, 0, 0)),
        ],
        out_specs=pl.BlockSpec((1, BLK, WIDE), lambda b, i: (b, i, 0)),
        out_shape=jax.ShapeDtypeStruct((2, N, WIDE), f32),
    )(x_src, w_src).reshape(2 * N, WIDE)

    sidx = jnp.stack([ei_user_spot[0], ei_spot_user[0] + N])
    didx = jnp.stack([ei_user_spot[1], ei_spot_user[1]])
    idx = jnp.stack([sidx, didx]).reshape(4 * NS, NBLK, KB, CHUNK)
    zros = jnp.zeros((CHUNK, WIDE), f32)

    mesh = plsc.VectorSubcoreMesh(core_axis_name="c", subcore_axis_name="s")
    acc = pl.kernel(
        _sc_body,
        out_type=jax.ShapeDtypeStruct((2, N_PAD, WIDE), f32),
        mesh=mesh,
        scratch_types=[
            pltpu.VMEM((KB, CHUNK), jnp.int32),
            pltpu.VMEM((KB, CHUNK), jnp.int32),
            pltpu.VMEM((CHUNK, WIDE), f32),
            pltpu.VMEM_SHARED((N_PAD, WIDE), f32),
            pltpu.SemaphoreType.DMA,
        ],
        compiler_params=pltpu.CompilerParams(use_tc_tiling_on_sc=False),
    )(table, idx, zros)

    x_tgt = jnp.stack([x_spot, x_user])
    w_tgt = jnp.stack([W_tgt_us, W_tgt_su])
    wih = jnp.stack([Wih_us, Wih_su])
    whh = jnp.stack([Whh_us, Whh_su])
    bih = jnp.stack([bih_us, bih_su]).reshape(2, 1, 3 * H)
    bhh = jnp.stack([bhh_us, bhh_su]).reshape(2, 1, 3 * H)
    out = pl.pallas_call(
        _gru_body,
        grid=(2, nb),
        in_specs=[
            pl.BlockSpec((1, BLK, DIN), lambda b, i: (b, i, 0)),
            pl.BlockSpec((1, H, DIN), lambda b, i: (b, 0, 0)),
            pl.BlockSpec((1, BLK, WIDE), lambda b, i: (b, i, 0)),
            pl.BlockSpec((1, 3 * H, H), lambda b, i: (b, 0, 0)),
            pl.BlockSpec((1, 3 * H, H), lambda b, i: (b, 0, 0)),
            pl.BlockSpec((1, 1, 3 * H), lambda b, i: (b, 0, 0)),
            pl.BlockSpec((1, 1, 3 * H), lambda b, i: (b, 0, 0)),
        ],
        out_specs=pl.BlockSpec((1, BLK, H), lambda b, i: (b, i, 0)),
        out_shape=jax.ShapeDtypeStruct((2, N, H), f32),
    )(x_tgt, w_tgt, acc[:, :N], wih, whh, bih, bhh)

    return (out[0], out[1])


# speedup vs baseline: 7.5701x; 1.0738x over previous
import jax
import jax.numpy as jnp
from jax import lax
from jax.experimental import pallas as pl
from jax.experimental.pallas import tpu as pltpu
from jax.experimental.pallas import tpu_sc as plsc

N = 10000
DIN = 128
H = 128
E = 640000
WIDE = 144
CNT = 128
N_PAD = 10240
NS = 16
CHUNK = 40
KB = 8
NBLK = 125
ROWS_PER_TILE = N_PAD // NS
BLK = 1000


def _proj_body(x_ref, w_ref, o_ref):
    x = x_ref[0]
    w = w_ref[0]
    xw = lax.dot_general(x, w, (((1,), (1,)), ((), ())),
                         preferred_element_type=jnp.float32)
    extra = jnp.where(
        lax.broadcasted_iota(jnp.int32, (BLK, WIDE - H), 1) == 0, 1.0, 0.0)
    o_ref[0] = jnp.concatenate([xw, extra], axis=1)


def _sc_body(table_hbm, idx_hbm, zros_hbm, out_hbm,
             sidx_v, didx_v, rows_v, acc_sh, sem):
    cid = lax.axis_index("c")
    tid = lax.axis_index("s")
    base = tid * ROWS_PER_TILE
    pltpu.sync_copy(zros_hbm, rows_v.at[0])
    for k in range(ROWS_PER_TILE // CHUNK):
        pltpu.sync_copy(rows_v.at[0], acc_sh.at[pl.ds(base + k * CHUNK, CHUNK)])
    plsc.subcore_barrier()
    ws = cid * NS + tid
    wd = 2 * NS + ws

    pltpu.sync_copy(idx_hbm.at[ws, 0], sidx_v.at[0])
    pltpu.sync_copy(idx_hbm.at[wd, 0], didx_v.at[0])
    pltpu.async_copy(table_hbm.at[sidx_v.at[0, 0]], rows_v.at[0], sem)

    def block(b, carry):
        slot = lax.rem(b, 2)
        nslot = lax.rem(b + 1, 2)

        @pl.when(b < NBLK - 1)
        def _():
            pltpu.sync_copy(idx_hbm.at[ws, b + 1], sidx_v.at[nslot])
            pltpu.sync_copy(idx_hbm.at[wd, b + 1], didx_v.at[nslot])

        for k in range(KB):
            buf = k % 2
            nbuf = 1 - buf
            pltpu.make_async_copy(
                table_hbm.at[sidx_v.at[slot, k]], rows_v.at[buf], sem).wait()
            if k < KB - 1:
                pltpu.async_copy(
                    table_hbm.at[sidx_v.at[slot, k + 1]], rows_v.at[nbuf], sem)
            else:
                @pl.when(b < NBLK - 1)
                def _():
                    pltpu.async_copy(
                        table_hbm.at[sidx_v.at[nslot, 0]], rows_v.at[nbuf], sem)
            pltpu.sync_copy(rows_v.at[buf], acc_sh.at[didx_v.at[slot, k]],
                            add=True)
        return carry

    lax.fori_loop(0, NBLK, block, 0, unroll=False)
    plsc.subcore_barrier()
    for k in range(ROWS_PER_TILE // CHUNK):
        sl = pl.ds(base + k * CHUNK, CHUNK)
        pltpu.sync_copy(acc_sh.at[sl], rows_v.at[0])
        pltpu.sync_copy(rows_v.at[0], out_hbm.at[cid].at[sl])


def _gru_body(x_ref, wt_ref, acc_ref, wih_ref, whh_ref, bih_ref, bhh_ref,
              o_ref):
    x = x_ref[0]
    tgt = lax.dot_general(x, wt_ref[0], (((1,), (1,)), ((), ())),
                          preferred_element_type=jnp.float32)
    acc = acc_ref[0]
    agg = acc[:, :H] / jnp.maximum(acc[:, CNT:CNT + 1], 1.0)
    gi = lax.dot_general(tgt, wih_ref[0], (((1,), (1,)), ((), ())),
                         preferred_element_type=jnp.float32) + bih_ref[0, 0]
    gh = lax.dot_general(agg, whh_ref[0], (((1,), (1,)), ((), ())),
                         preferred_element_type=jnp.float32) + bhh_ref[0, 0]
    r = jax.nn.sigmoid(gi[:, :H] + gh[:, :H])
    z = jax.nn.sigmoid(gi[:, H:2 * H] + gh[:, H:2 * H])
    n = jnp.tanh(gi[:, 2 * H:] + r * gh[:, 2 * H:])
    o_ref[0] = jax.nn.relu((1.0 - z) * n + z * agg)


def kernel(x_spot, x_user, ei_user_spot, ei_spot_user,
           W_src_us, W_tgt_us, W_src_su, W_tgt_su,
           Wih_us, Whh_us, bih_us, bhh_us,
           Wih_su, Whh_su, bih_su, bhh_su):
    f32 = jnp.float32
    nb = N // BLK

    x_src = jnp.stack([x_user, x_spot])
    w_src = jnp.stack([W_src_us, W_src_su])
    table = pl.pallas_call(
        _proj_body,
        grid=(2, nb),
        in_specs=[
            pl.BlockSpec((1, BLK, DIN), lambda b, i: (b, i, 0)),
            pl.BlockSpec((1, H, DIN), lambda b, i: (b, 0, 0)),
        ],
        out_specs=pl.BlockSpec((1, BLK, WIDE), lambda b, i: (b, i, 0)),
        out_shape=jax.ShapeDtypeStruct((2, N, WIDE), f32),
    )(x_src, w_src).reshape(2 * N, WIDE)

    sidx = jnp.stack([ei_user_spot[0], ei_spot_user[0] + N])
    didx = jnp.stack([ei_user_spot[1], ei_spot_user[1]])
    idx = jnp.stack([sidx, didx]).reshape(4 * NS, NBLK, KB, CHUNK)
    zros = jnp.zeros((CHUNK, WIDE), f32)

    mesh = plsc.VectorSubcoreMesh(core_axis_name="c", subcore_axis_name="s")
    acc = pl.kernel(
        _sc_body,
        out_type=jax.ShapeDtypeStruct((2, N_PAD, WIDE), f32),
        mesh=mesh,
        scratch_types=[
            pltpu.VMEM((2, KB, CHUNK), jnp.int32),
            pltpu.VMEM((2, KB, CHUNK), jnp.int32),
            pltpu.VMEM((2, CHUNK, WIDE), f32),
            pltpu.VMEM_SHARED((N_PAD, WIDE), f32),
            pltpu.SemaphoreType.DMA,
        ],
        compiler_params=pltpu.CompilerParams(use_tc_tiling_on_sc=False),
    )(table, idx, zros)

    x_tgt = jnp.stack([x_spot, x_user])
    w_tgt = jnp.stack([W_tgt_us, W_tgt_su])
    wih = jnp.stack([Wih_us, Wih_su])
    whh = jnp.stack([Whh_us, Whh_su])
    bih = jnp.stack([bih_us, bih_su]).reshape(2, 1, 3 * H)
    bhh = jnp.stack([bhh_us, bhh_su]).reshape(2, 1, 3 * H)
    out = pl.pallas_call(
        _gru_body,
        grid=(2, nb),
        in_specs=[
            pl.BlockSpec((1, BLK, DIN), lambda b, i: (b, i, 0)),
            pl.BlockSpec((1, H, DIN), lambda b, i: (b, 0, 0)),
            pl.BlockSpec((1, BLK, WIDE), lambda b, i: (b, i, 0)),
            pl.BlockSpec((1, 3 * H, H), lambda b, i: (b, 0, 0)),
            pl.BlockSpec((1, 3 * H, H), lambda b, i: (b, 0, 0)),
            pl.BlockSpec((1, 1, 3 * H), lambda b, i: (b, 0, 0)),
            pl.BlockSpec((1, 1, 3 * H), lambda b, i: (b, 0, 0)),
        ],
        out_specs=pl.BlockSpec((1, BLK, H), lambda b, i: (b, i, 0)),
        out_shape=jax.ShapeDtypeStruct((2, N, H), f32),
    )(x_tgt, w_tgt, acc[:, :N], wih, whh, bih, bhh)

    return (out[0], out[1])


# speedup vs baseline: 11.9587x; 1.5797x over previous
import jax
import jax.numpy as jnp
from jax import lax
from jax.experimental import pallas as pl
from jax.experimental.pallas import tpu as pltpu
from jax.experimental.pallas import tpu_sc as plsc

N = 10000
DIN = 128
H = 128
E = 640000
W = 80
CNT = 64
N_PAD = 10240
NS = 16
CHUNK = 128
KB = 4
NBLK = 79
NCHUNK = KB * NBLK
E_PAD = NS * NCHUNK * CHUNK
RB = 4
ROWS_PER_TILE = N_PAD // NS
BLK = 1000


def _proj_body(x_ref, w_ref, oa_ref, ob_ref):
    x = x_ref[0]
    w = w_ref[0]
    xw = lax.dot_general(x, w, (((1,), (1,)), ((), ())),
                         preferred_element_type=jnp.float32)
    extra = jnp.where(
        lax.broadcasted_iota(jnp.int32, (BLK, W - CNT), 1) == 0, 1.0, 0.0)
    oa_ref[0] = jnp.concatenate([xw[:, :CNT], extra], axis=1)
    ob_ref[0] = jnp.concatenate(
        [xw[:, CNT:], jnp.zeros((BLK, W - CNT), jnp.float32)], axis=1)


def _sc_body(ta_hbm, tb_hbm, idx_hbm, zros_hbm, out_hbm,
             sidx_v, didx_v, rows_v, acc_sh, gsem, ssem):
    cid = lax.axis_index("c")
    tid = lax.axis_index("s")
    base = tid * ROWS_PER_TILE
    ws = cid * NS + tid
    wd = 2 * NS + ws

    def wait_g():
        pltpu.make_async_copy(zros_hbm, rows_v.at[0], gsem).wait()

    def wait_s():
        pltpu.make_async_copy(zros_hbm, rows_v.at[0], ssem).wait()

    def zero_own_rows():
        pltpu.sync_copy(zros_hbm, rows_v.at[0])
        for k in range(ROWS_PER_TILE // CHUNK):
            pltpu.sync_copy(rows_v.at[0],
                            acc_sh.at[pl.ds(base + k * CHUNK, CHUNK)])

    zero_own_rows()
    plsc.subcore_barrier()

    for p, table in enumerate((ta_hbm, tb_hbm)):
        pltpu.sync_copy(idx_hbm.at[ws, 0], sidx_v.at[0])
        pltpu.sync_copy(idx_hbm.at[wd, 0], didx_v.at[0])
        pltpu.async_copy(table.at[sidx_v.at[0, 0]], rows_v.at[0], gsem)
        pltpu.async_copy(table.at[sidx_v.at[0, 1]], rows_v.at[1], gsem)
        pltpu.sync_copy(idx_hbm.at[ws, 1], sidx_v.at[1])
        pltpu.sync_copy(idx_hbm.at[wd, 1], didx_v.at[1])
        for k in range(KB):
            wait_g()
            if k >= 2:
                wait_s()
            src_slot, src_entry = (0, k + 2) if k < 2 else (1, k - 2)
            pltpu.async_copy(table.at[sidx_v.at[src_slot, src_entry]],
                             rows_v.at[(k + 2) % RB], gsem)
            pltpu.async_copy(rows_v.at[k], acc_sh.at[didx_v.at[0, k]],
                             ssem, add=True)

        def block(b, carry):
            slot = lax.rem(b, 2)
            nslot = lax.rem(b + 1, 2)

            @pl.when(b < NBLK - 1)
            def _():
                pltpu.sync_copy(idx_hbm.at[ws, b + 1], sidx_v.at[nslot])
                pltpu.sync_copy(idx_hbm.at[wd, b + 1], didx_v.at[nslot])

            for k in range(KB):
                wait_g()
                wait_s()
                if k < 2:
                    pltpu.async_copy(table.at[sidx_v.at[slot, k + 2]],
                                     rows_v.at[k + 2], gsem)
                else:
                    @pl.when(b < NBLK - 1)
                    def _():
                        pltpu.async_copy(table.at[sidx_v.at[nslot, k - 2]],
                                         rows_v.at[k - 2], gsem)
                pltpu.async_copy(rows_v.at[k], acc_sh.at[didx_v.at[slot, k]],
                                 ssem, add=True)
            return carry

        lax.fori_loop(1, NBLK, block, 0, unroll=False)
        wait_s()
        wait_s()
        plsc.subcore_barrier()
        for k in range(ROWS_PER_TILE // CHUNK):
            sl = pl.ds(base + k * CHUNK, CHUNK)
            pltpu.sync_copy(acc_sh.at[sl], rows_v.at[0])
            pltpu.sync_copy(rows_v.at[0], out_hbm.at[cid, p].at[sl])
        if p == 0:
            zero_own_rows()
            plsc.subcore_barrier()


def _gru_body(x_ref, wt_ref, acca_ref, accb_ref, wih_ref, whh_ref,
              bih_ref, bhh_ref, o_ref):
    x = x_ref[0]
    tgt = lax.dot_general(x, wt_ref[0], (((1,), (1,)), ((), ())),
                          preferred_element_type=jnp.float32)
    acca = acca_ref[0]
    aggsum = jnp.concatenate([acca[:, :CNT], accb_ref[0][:, :CNT]], axis=1)
    agg = aggsum / jnp.maximum(acca[:, CNT:CNT + 1], 1.0)
    gi = lax.dot_general(tgt, wih_ref[0], (((1,), (1,)), ((), ())),
                         preferred_element_type=jnp.float32) + bih_ref[0, 0]
    gh = lax.dot_general(agg, whh_ref[0], (((1,), (1,)), ((), ())),
                         preferred_element_type=jnp.float32) + bhh_ref[0, 0]
    r = jax.nn.sigmoid(gi[:, :H] + gh[:, :H])
    z = jax.nn.sigmoid(gi[:, H:2 * H] + gh[:, H:2 * H])
    n = jnp.tanh(gi[:, 2 * H:] + r * gh[:, 2 * H:])
    o_ref[0] = jax.nn.relu((1.0 - z) * n + z * agg)


def kernel(x_spot, x_user, ei_user_spot, ei_spot_user,
           W_src_us, W_tgt_us, W_src_su, W_tgt_su,
           Wih_us, Whh_us, bih_us, bhh_us,
           Wih_su, Whh_su, bih_su, bhh_su):
    f32 = jnp.float32
    nb = N // BLK

    x_src = jnp.stack([x_user, x_spot])
    w_src = jnp.stack([W_src_us, W_src_su])
    table_a, table_b = pl.pallas_call(
        _proj_body,
        grid=(2, nb),
        in_specs=[
            pl.BlockSpec((1, BLK, DIN), lambda b, i: (b, i, 0)),
            pl.BlockSpec((1, H, DIN), lambda b, i: (b, 0, 0)),
        ],
        out_specs=[
            pl.BlockSpec((1, BLK, W), lambda b, i: (b, i, 0)),
            pl.BlockSpec((1, BLK, W), lambda b, i: (b, i, 0)),
        ],
        out_shape=[
            jax.ShapeDtypeStruct((2, N, W), f32),
            jax.ShapeDtypeStruct((2, N, W), f32),
        ],
    )(x_src, w_src)
    table_a = table_a.reshape(2 * N, W)
    table_b = table_b.reshape(2 * N, W)

    pad = E_PAD - E
    ar = jnp.arange(pad, dtype=jnp.int32)
    pad_src = ar % (2 * N)
    pad_dst = N + ar % (N_PAD - N)
    sidx = jnp.stack([
        jnp.concatenate([ei_user_spot[0], pad_src]),
        jnp.concatenate([ei_spot_user[0] + N, pad_src]),
    ])
    didx = jnp.stack([
        jnp.concatenate([ei_user_spot[1], pad_dst]),
        jnp.concatenate([ei_spot_user[1], pad_dst]),
    ])
    idx = jnp.stack([sidx, didx]).reshape(4 * NS, NBLK, KB, CHUNK)
    zros = jnp.zeros((CHUNK, W), f32)

    mesh = plsc.VectorSubcoreMesh(core_axis_name="c", subcore_axis_name="s")
    acc = pl.kernel(
        _sc_body,
        out_type=jax.ShapeDtypeStruct((2, 2, N_PAD, W), f32),
        mesh=mesh,
        scratch_types=[
            pltpu.VMEM((2, KB, CHUNK), jnp.int32),
            pltpu.VMEM((2, KB, CHUNK), jnp.int32),
            pltpu.VMEM((RB, CHUNK, W), f32),
            pltpu.VMEM_SHARED((N_PAD, W), f32),
            pltpu.SemaphoreType.DMA,
            pltpu.SemaphoreType.DMA,
        ],
        compiler_params=pltpu.CompilerParams(use_tc_tiling_on_sc=False),
    )(table_a, table_b, idx, zros)

    x_tgt = jnp.stack([x_spot, x_user])
    w_tgt = jnp.stack([W_tgt_us, W_tgt_su])
    wih = jnp.stack([Wih_us, Wih_su])
    whh = jnp.stack([Whh_us, Whh_su])
    bih = jnp.stack([bih_us, bih_su]).reshape(2, 1, 3 * H)
    bhh = jnp.stack([bhh_us, bhh_su]).reshape(2, 1, 3 * H)
    out = pl.pallas_call(
        _gru_body,
        grid=(2, nb),
        in_specs=[
            pl.BlockSpec((1, BLK, DIN), lambda b, i: (b, i, 0)),
            pl.BlockSpec((1, H, DIN), lambda b, i: (b, 0, 0)),
            pl.BlockSpec((1, BLK, W), lambda b, i: (b, i, 0)),
            pl.BlockSpec((1, BLK, W), lambda b, i: (b, i, 0)),
            pl.BlockSpec((1, 3 * H, H), lambda b, i: (b, 0, 0)),
            pl.BlockSpec((1, 3 * H, H), lambda b, i: (b, 0, 0)),
            pl.BlockSpec((1, 1, 3 * H), lambda b, i: (b, 0, 0)),
            pl.BlockSpec((1, 1, 3 * H), lambda b, i: (b, 0, 0)),
        ],
        out_specs=pl.BlockSpec((1, BLK, H), lambda b, i: (b, i, 0)),
        out_shape=jax.ShapeDtypeStruct((2, N, H), f32),
    )(x_tgt, w_tgt, acc[:, 0, :N], acc[:, 1, :N], wih, whh, bih, bhh)

    return (out[0], out[1])


# speedup vs baseline: 14.4779x; 1.2107x over previous
import jax
import jax.numpy as jnp
from jax import lax
from jax.experimental import pallas as pl
from jax.experimental.pallas import tpu as pltpu
from jax.experimental.pallas import tpu_sc as plsc

N = 10000
DIN = 128
H = 128
E = 640000
W = 80
CNT = 64
N_PAD = 10240
NS = 16
CHUNK = 128
KB = 4
NBLK = 79
NCHUNK = KB * NBLK
E_PAD = NS * NCHUNK * CHUNK
RB = 5
ROWS_PER_TILE = N_PAD // NS
BLK = 1000


def _proj_body(x_ref, w_ref, oa_ref, ob_ref):
    x = x_ref[0]
    w = w_ref[0]
    xw = lax.dot_general(x, w, (((1,), (1,)), ((), ())),
                         preferred_element_type=jnp.float32)
    extra = jnp.where(
        lax.broadcasted_iota(jnp.int32, (BLK, W - CNT), 1) == 0, 1.0, 0.0)
    oa_ref[0] = jnp.concatenate([xw[:, :CNT], extra], axis=1)
    ob_ref[0] = jnp.concatenate(
        [xw[:, CNT:], jnp.zeros((BLK, W - CNT), jnp.float32)], axis=1)


def _sc_body(ta_hbm, tb_hbm, idx_hbm, zros_hbm, out_hbm,
             sidx_v, didx_v, rows_v, acc_sh, gsem, ssem, isem):
    cid = lax.axis_index("c")
    tid = lax.axis_index("s")
    base = tid * ROWS_PER_TILE
    ws = cid * NS + tid
    wd = 2 * NS + ws

    def wait_g():
        pltpu.make_async_copy(zros_hbm, rows_v.at[0], gsem).wait()

    def wait_s():
        pltpu.make_async_copy(zros_hbm, rows_v.at[0], ssem).wait()

    def wait_i():
        pltpu.make_async_copy(idx_hbm.at[0, 0], sidx_v.at[0], isem).wait()
        pltpu.make_async_copy(idx_hbm.at[0, 0], sidx_v.at[0], isem).wait()

    def stage(b, slot):
        pltpu.async_copy(idx_hbm.at[ws, b], sidx_v.at[slot], isem)
        pltpu.async_copy(idx_hbm.at[wd, b], didx_v.at[slot], isem)

    def zero_own_rows():
        pltpu.sync_copy(zros_hbm, rows_v.at[0])
        for k in range(ROWS_PER_TILE // CHUNK):
            pltpu.sync_copy(rows_v.at[0],
                            acc_sh.at[pl.ds(base + k * CHUNK, CHUNK)])

    zero_own_rows()
    plsc.subcore_barrier()

    for p, table in enumerate((ta_hbm, tb_hbm)):
        pltpu.sync_copy(idx_hbm.at[ws, 0], sidx_v.at[0])
        pltpu.sync_copy(idx_hbm.at[wd, 0], didx_v.at[0])
        for k in range(3):
            pltpu.async_copy(table.at[sidx_v.at[0, k]], rows_v.at[k], gsem)
        stage(1, 1)
        for k in range(KB):
            wait_g()
            if k == 1:
                wait_i()
            if k >= 2:
                wait_s()
            src_slot, src_entry = (0, 3) if k == 0 else (1, k - 1)
            pltpu.async_copy(table.at[sidx_v.at[src_slot, src_entry]],
                             rows_v.at[(k + 3) % RB], gsem)
            pltpu.async_copy(rows_v.at[k], acc_sh.at[didx_v.at[0, k]],
                             ssem, add=True)

        def block(b, carry):
            slot = lax.rem(b, 3)
            nslot = lax.rem(b + 1, 3)
            last = b >= NBLK - 1

            @pl.when(jnp.logical_not(last))
            def _():
                stage(b + 1, nslot)

            for k in range(KB):
                i = b * KB + k
                buf = lax.rem(i, RB)
                nbuf = lax.rem(i + 3, RB)
                wait_g()
                if k == 1:
                    @pl.when(jnp.logical_not(last))
                    def _():
                        wait_i()
                wait_s()
                if k == 0:
                    pltpu.async_copy(table.at[sidx_v.at[slot, 3]],
                                     rows_v.at[nbuf], gsem)
                else:
                    @pl.when(jnp.logical_not(last))
                    def _():
                        pltpu.async_copy(table.at[sidx_v.at[nslot, k - 1]],
                                         rows_v.at[nbuf], gsem)
                pltpu.async_copy(rows_v.at[buf],
                                 acc_sh.at[didx_v.at[slot, k]],
                                 ssem, add=True)
            return carry

        lax.fori_loop(1, NBLK, block, 0, unroll=False)
        wait_s()
        wait_s()
        plsc.subcore_barrier()
        for k in range(ROWS_PER_TILE // CHUNK):
            sl = pl.ds(base + k * CHUNK, CHUNK)
            pltpu.sync_copy(acc_sh.at[sl], rows_v.at[0])
            pltpu.sync_copy(rows_v.at[0], out_hbm.at[cid, p].at[sl])
        if p == 0:
            zero_own_rows()
            plsc.subcore_barrier()


def _gru_body(x_ref, wt_ref, acca_ref, accb_ref, wih_ref, whh_ref,
              bih_ref, bhh_ref, o_ref):
    x = x_ref[0]
    tgt = lax.dot_general(x, wt_ref[0], (((1,), (1,)), ((), ())),
                          preferred_element_type=jnp.float32)
    acca = acca_ref[0]
    aggsum = jnp.concatenate([acca[:, :CNT], accb_ref[0][:, :CNT]], axis=1)
    agg = aggsum / jnp.maximum(acca[:, CNT:CNT + 1], 1.0)
    gi = lax.dot_general(tgt, wih_ref[0], (((1,), (1,)), ((), ())),
                         preferred_element_type=jnp.float32) + bih_ref[0, 0]
    gh = lax.dot_general(agg, whh_ref[0], (((1,), (1,)), ((), ())),
                         preferred_element_type=jnp.float32) + bhh_ref[0, 0]
    r = jax.nn.sigmoid(gi[:, :H] + gh[:, :H])
    z = jax.nn.sigmoid(gi[:, H:2 * H] + gh[:, H:2 * H])
    n = jnp.tanh(gi[:, 2 * H:] + r * gh[:, 2 * H:])
    o_ref[0] = jax.nn.relu((1.0 - z) * n + z * agg)


def kernel(x_spot, x_user, ei_user_spot, ei_spot_user,
           W_src_us, W_tgt_us, W_src_su, W_tgt_su,
           Wih_us, Whh_us, bih_us, bhh_us,
           Wih_su, Whh_su, bih_su, bhh_su):
    f32 = jnp.float32
    nb = N // BLK

    x_src = jnp.stack([x_user, x_spot])
    w_src = jnp.stack([W_src_us, W_src_su])
    table_a, table_b = pl.pallas_call(
        _proj_body,
        grid=(2, nb),
        in_specs=[
            pl.BlockSpec((1, BLK, DIN), lambda b, i: (b, i, 0)),
            pl.BlockSpec((1, H, DIN), lambda b, i: (b, 0, 0)),
        ],
        out_specs=[
            pl.BlockSpec((1, BLK, W), lambda b, i: (b, i, 0)),
            pl.BlockSpec((1, BLK, W), lambda b, i: (b, i, 0)),
        ],
        out_shape=[
            jax.ShapeDtypeStruct((2, N, W), f32),
            jax.ShapeDtypeStruct((2, N, W), f32),
        ],
    )(x_src, w_src)
    table_a = table_a.reshape(2 * N, W)
    table_b = table_b.reshape(2 * N, W)

    pad = E_PAD - E
    ar = jnp.arange(pad, dtype=jnp.int32)
    pad_src = ar % (2 * N)
    pad_dst = N + ar % (N_PAD - N)
    sidx = jnp.stack([
        jnp.concatenate([ei_user_spot[0], pad_src]),
        jnp.concatenate([ei_spot_user[0] + N, pad_src]),
    ])
    didx = jnp.stack([
        jnp.concatenate([ei_user_spot[1], pad_dst]),
        jnp.concatenate([ei_spot_user[1], pad_dst]),
    ])
    idx = jnp.stack([sidx, didx]).reshape(4 * NS, NBLK, KB, CHUNK)
    zros = jnp.zeros((CHUNK, W), f32)

    mesh = plsc.VectorSubcoreMesh(core_axis_name="c", subcore_axis_name="s")
    acc = pl.kernel(
        _sc_body,
        out_type=jax.ShapeDtypeStruct((2, 2, N_PAD, W), f32),
        mesh=mesh,
        scratch_types=[
            pltpu.VMEM((3, KB, CHUNK), jnp.int32),
            pltpu.VMEM((3, KB, CHUNK), jnp.int32),
            pltpu.VMEM((RB, CHUNK, W), f32),
            pltpu.VMEM_SHARED((N_PAD, W), f32),
            pltpu.SemaphoreType.DMA,
            pltpu.SemaphoreType.DMA,
            pltpu.SemaphoreType.DMA,
        ],
        compiler_params=pltpu.CompilerParams(use_tc_tiling_on_sc=False),
    )(table_a, table_b, idx, zros)

    x_tgt = jnp.stack([x_spot, x_user])
    w_tgt = jnp.stack([W_tgt_us, W_tgt_su])
    wih = jnp.stack([Wih_us, Wih_su])
    whh = jnp.stack([Whh_us, Whh_su])
    bih = jnp.stack([bih_us, bih_su]).reshape(2, 1, 3 * H)
    bhh = jnp.stack([bhh_us, bhh_su]).reshape(2, 1, 3 * H)
    out = pl.pallas_call(
        _gru_body,
        grid=(2, nb),
        in_specs=[
            pl.BlockSpec((1, BLK, DIN), lambda b, i: (b, i, 0)),
            pl.BlockSpec((1, H, DIN), lambda b, i: (b, 0, 0)),
            pl.BlockSpec((1, BLK, W), lambda b, i: (b, i, 0)),
            pl.BlockSpec((1, BLK, W), lambda b, i: (b, i, 0)),
            pl.BlockSpec((1, 3 * H, H), lambda b, i: (b, 0, 0)),
            pl.BlockSpec((1, 3 * H, H), lambda b, i: (b, 0, 0)),
            pl.BlockSpec((1, 1, 3 * H), lambda b, i: (b, 0, 0)),
            pl.BlockSpec((1, 1, 3 * H), lambda b, i: (b, 0, 0)),
        ],
        out_specs=pl.BlockSpec((1, BLK, H), lambda b, i: (b, i, 0)),
        out_shape=jax.ShapeDtypeStruct((2, N, H), f32),
    )(x_tgt, w_tgt, acc[:, 0, :N], acc[:, 1, :N], wih, whh, bih, bhh)

    return (out[0], out[1])


# speedup vs baseline: 15.4812x; 1.0693x over previous
import jax
import jax.numpy as jnp
from jax import lax
from jax.experimental import pallas as pl
from jax.experimental.pallas import tpu as pltpu
from jax.experimental.pallas import tpu_sc as plsc

N = 10000
DIN = 128
H = 128
E = 640000
W = 64
WC = 16
N_PAD = 10240
NS = 16
CHUNK = 128
KB = 4
NBLK = 79
NCHUNK = KB * NBLK
E_PAD = NS * NCHUNK * CHUNK
RB = 5
ROWS_PER_TILE = N_PAD // NS
BLK = 1000


def _proj_body(x_ref, w_ref, oa_ref, ob_ref):
    x = x_ref[0]
    w = w_ref[0]
    xw = lax.dot_general(x, w, (((1,), (1,)), ((), ())),
                         preferred_element_type=jnp.float32)
    oa_ref[0] = xw[:, :W]
    ob_ref[0] = xw[:, W:]


def _sc_body(ta_hbm, tb_hbm, idx_hbm, zros_hbm, zrosc_hbm, ones_hbm,
             out_hbm, outc_hbm,
             sidx_v, didx_v, rows_v, ones_v, acc_sh, cnt_sh,
             gsem, ssem, isem, csem):
    cid = lax.axis_index("c")
    tid = lax.axis_index("s")
    base = tid * ROWS_PER_TILE
    ws = cid * NS + tid
    wd = 2 * NS + ws

    def wait_g():
        pltpu.make_async_copy(zros_hbm, rows_v.at[0], gsem).wait()

    def wait_s():
        pltpu.make_async_copy(zros_hbm, rows_v.at[0], ssem).wait()

    def wait_i():
        pltpu.make_async_copy(idx_hbm.at[0, 0], sidx_v.at[0], isem).wait()
        pltpu.make_async_copy(idx_hbm.at[0, 0], sidx_v.at[0], isem).wait()

    def stage(b, slot):
        pltpu.async_copy(idx_hbm.at[ws, b], sidx_v.at[slot], isem)
        pltpu.async_copy(idx_hbm.at[wd, b], didx_v.at[slot], isem)

    def zero_own_rows():
        pltpu.sync_copy(zros_hbm, rows_v.at[0])
        for k in range(ROWS_PER_TILE // CHUNK):
            pltpu.sync_copy(rows_v.at[0],
                            acc_sh.at[pl.ds(base + k * CHUNK, CHUNK)])

    zero_own_rows()
    pltpu.sync_copy(zrosc_hbm, ones_v)
    for k in range(ROWS_PER_TILE // CHUNK):
        pltpu.sync_copy(ones_v, cnt_sh.at[pl.ds(base + k * CHUNK, CHUNK)])
    pltpu.sync_copy(ones_hbm, ones_v)
    plsc.subcore_barrier()

    for p, table in enumerate((ta_hbm, tb_hbm)):
        pltpu.sync_copy(idx_hbm.at[ws, 0], sidx_v.at[0])
        pltpu.sync_copy(idx_hbm.at[wd, 0], didx_v.at[0])
        for k in range(3):
            pltpu.async_copy(table.at[sidx_v.at[0, k]], rows_v.at[k], gsem)
        stage(1, 1)
        for k in range(KB):
            wait_g()
            if k == 1:
                wait_i()
            if k >= 2:
                wait_s()
            src_slot, src_entry = (0, 3) if k == 0 else (1, k - 1)
            pltpu.async_copy(table.at[sidx_v.at[src_slot, src_entry]],
                             rows_v.at[(k + 3) % RB], gsem)
            pltpu.async_copy(rows_v.at[k], acc_sh.at[didx_v.at[0, k]],
                             ssem, add=True)
            if p == 0:
                pltpu.async_copy(ones_v, cnt_sh.at[didx_v.at[0, k]],
                                 csem, add=True)

        def block(b, carry):
            slot = lax.rem(b, 3)
            nslot = lax.rem(b + 1, 3)
            last = b >= NBLK - 1

            @pl.when(jnp.logical_not(last))
            def _():
                stage(b + 1, nslot)

            for k in range(KB):
                i = b * KB + k
                buf = lax.rem(i, RB)
                nbuf = lax.rem(i + 3, RB)
                wait_g()
                if k == 1:
                    @pl.when(jnp.logical_not(last))
                    def _():
                        wait_i()
                wait_s()
                if k == 0:
                    pltpu.async_copy(table.at[sidx_v.at[slot, 3]],
                                     rows_v.at[nbuf], gsem)
                else:
                    @pl.when(jnp.logical_not(last))
                    def _():
                        pltpu.async_copy(table.at[sidx_v.at[nslot, k - 1]],
                                         rows_v.at[nbuf], gsem)
                pltpu.async_copy(rows_v.at[buf],
                                 acc_sh.at[didx_v.at[slot, k]],
                                 ssem, add=True)
                if p == 0:
                    pltpu.async_copy(ones_v,
                                     cnt_sh.at[didx_v.at[slot, k]],
                                     csem, add=True)
            return carry

        lax.fori_loop(1, NBLK, block, 0, unroll=False)
        wait_s()
        wait_s()
        if p == 0:
            def drain(_, carry):
                pltpu.make_async_copy(ones_hbm, ones_v, csem).wait()
                return carry
            lax.fori_loop(0, NCHUNK, drain, 0, unroll=False)
        plsc.subcore_barrier()
        for k in range(ROWS_PER_TILE // CHUNK):
            sl = pl.ds(base + k * CHUNK, CHUNK)
            pltpu.sync_copy(acc_sh.at[sl], rows_v.at[0])
            pltpu.sync_copy(rows_v.at[0], out_hbm.at[cid, p].at[sl])
        if p == 0:
            for k in range(ROWS_PER_TILE // CHUNK):
                sl = pl.ds(base + k * CHUNK, CHUNK)
                pltpu.sync_copy(cnt_sh.at[sl], ones_v)
                pltpu.sync_copy(ones_v, outc_hbm.at[cid].at[sl])
            zero_own_rows()
            pltpu.sync_copy(ones_hbm, ones_v)
            plsc.subcore_barrier()


def _gru_body(x_ref, wt_ref, acca_ref, accb_ref, cnt_ref, wih_ref, whh_ref,
              bih_ref, bhh_ref, o_ref):
    x = x_ref[0]
    tgt = lax.dot_general(x, wt_ref[0], (((1,), (1,)), ((), ())),
                          preferred_element_type=jnp.float32)
    aggsum = jnp.concatenate([acca_ref[0], accb_ref[0]], axis=1)
    agg = aggsum / jnp.maximum(cnt_ref[0][:, :1], 1.0)
    gi = lax.dot_general(tgt, wih_ref[0], (((1,), (1,)), ((), ())),
                         preferred_element_type=jnp.float32) + bih_ref[0, 0]
    gh = lax.dot_general(agg, whh_ref[0], (((1,), (1,)), ((), ())),
                         preferred_element_type=jnp.float32) + bhh_ref[0, 0]
    r = jax.nn.sigmoid(gi[:, :H] + gh[:, :H])
    z = jax.nn.sigmoid(gi[:, H:2 * H] + gh[:, H:2 * H])
    n = jnp.tanh(gi[:, 2 * H:] + r * gh[:, 2 * H:])
    o_ref[0] = jax.nn.relu((1.0 - z) * n + z * agg)


def kernel(x_spot, x_user, ei_user_spot, ei_spot_user,
           W_src_us, W_tgt_us, W_src_su, W_tgt_su,
           Wih_us, Whh_us, bih_us, bhh_us,
           Wih_su, Whh_su, bih_su, bhh_su):
    f32 = jnp.float32
    nb = N // BLK

    x_src = jnp.stack([x_user, x_spot])
    w_src = jnp.stack([W_src_us, W_src_su])
    table_a, table_b = pl.pallas_call(
        _proj_body,
        grid=(2, nb),
        in_specs=[
            pl.BlockSpec((1, BLK, DIN), lambda b, i: (b, i, 0)),
            pl.BlockSpec((1, H, DIN), lambda b, i: (b, 0, 0)),
        ],
        out_specs=[
            pl.BlockSpec((1, BLK, W), lambda b, i: (b, i, 0)),
            pl.BlockSpec((1, BLK, W), lambda b, i: (b, i, 0)),
        ],
        out_shape=[
            jax.ShapeDtypeStruct((2, N, W), f32),
            jax.ShapeDtypeStruct((2, N, W), f32),
        ],
    )(x_src, w_src)
    table_a = table_a.reshape(2 * N, W)
    table_b = table_b.reshape(2 * N, W)

    pad = E_PAD - E
    ar = jnp.arange(pad, dtype=jnp.int32)
    pad_src = ar % (2 * N)
    pad_dst = N + ar % (N_PAD - N)
    sidx = jnp.stack([
        jnp.concatenate([ei_user_spot[0], pad_src]),
        jnp.concatenate([ei_spot_user[0] + N, pad_src]),
    ])
    didx = jnp.stack([
        jnp.concatenate([ei_user_spot[1], pad_dst]),
        jnp.concatenate([ei_spot_user[1], pad_dst]),
    ])
    idx = jnp.stack([sidx, didx]).reshape(4 * NS, NBLK, KB, CHUNK)
    zros = jnp.zeros((CHUNK, W), f32)
    zrosc = jnp.zeros((CHUNK, WC), f32)
    ones = jnp.ones((CHUNK, WC), f32)

    mesh = plsc.VectorSubcoreMesh(core_axis_name="c", subcore_axis_name="s")
    acc, cnt = pl.kernel(
        _sc_body,
        out_type=(
            jax.ShapeDtypeStruct((2, 2, N_PAD, W), f32),
            jax.ShapeDtypeStruct((2, N_PAD, WC), f32),
        ),
        mesh=mesh,
        scratch_types=[
            pltpu.VMEM((3, KB, CHUNK), jnp.int32),
            pltpu.VMEM((3, KB, CHUNK), jnp.int32),
            pltpu.VMEM((RB, CHUNK, W), f32),
            pltpu.VMEM((CHUNK, WC), f32),
            pltpu.VMEM_SHARED((N_PAD, W), f32),
            pltpu.VMEM_SHARED((N_PAD, WC), f32),
            pltpu.SemaphoreType.DMA,
            pltpu.SemaphoreType.DMA,
            pltpu.SemaphoreType.DMA,
            pltpu.SemaphoreType.DMA,
        ],
        compiler_params=pltpu.CompilerParams(use_tc_tiling_on_sc=False),
    )(table_a, table_b, idx, zros, zrosc, ones)

    x_tgt = jnp.stack([x_spot, x_user])
    w_tgt = jnp.stack([W_tgt_us, W_tgt_su])
    wih = jnp.stack([Wih_us, Wih_su])
    whh = jnp.stack([Whh_us, Whh_su])
    bih = jnp.stack([bih_us, bih_su]).reshape(2, 1, 3 * H)
    bhh = jnp.stack([bhh_us, bhh_su]).reshape(2, 1, 3 * H)
    out = pl.pallas_call(
        _gru_body,
        grid=(2, nb),
        in_specs=[
            pl.BlockSpec((1, BLK, DIN), lambda b, i: (b, i, 0)),
            pl.BlockSpec((1, H, DIN), lambda b, i: (b, 0, 0)),
            pl.BlockSpec((1, BLK, W), lambda b, i: (b, i, 0)),
            pl.BlockSpec((1, BLK, W), lambda b, i: (b, i, 0)),
            pl.BlockSpec((1, BLK, WC), lambda b, i: (b, i, 0)),
            pl.BlockSpec((1, 3 * H, H), lambda b, i: (b, 0, 0)),
            pl.BlockSpec((1, 3 * H, H), lambda b, i: (b, 0, 0)),
            pl.BlockSpec((1, 1, 3 * H), lambda b, i: (b, 0, 0)),
            pl.BlockSpec((1, 1, 3 * H), lambda b, i: (b, 0, 0)),
        ],
        out_specs=pl.BlockSpec((1, BLK, H), lambda b, i: (b, i, 0)),
        out_shape=jax.ShapeDtypeStruct((2, N, H), f32),
    )(x_tgt, w_tgt, acc[:, 0, :N], acc[:, 1, :N], cnt[:, :N],
      wih, whh, bih, bhh)

    return (out[0], out[1])
